# baseline (speedup 1.0000x reference)
import functools

import jax
import jax.numpy as jnp
from jax import lax
from jax.experimental import pallas as pl
from jax.experimental.pallas import tpu as pltpu
from jax.experimental.pallas import tpu_sc as plsc

LANES = 16
KPAD = 32


def _make_sc_dots(B, K, E):
    info = plsc.get_sparse_core_info()
    NW = info.num_cores * info.num_subcores
    rows_per_w = B // NW
    C = 64
    n_chunks = rows_per_w // C
    EV = E // LANES
    IDX_BLK = 128

    mesh = plsc.VectorSubcoreMesh(core_axis_name="c", subcore_axis_name="s")

    @functools.partial(
        pl.kernel,
        out_type=[
            jax.ShapeDtypeStruct((B,), jnp.float32),
            jax.ShapeDtypeStruct((B * KPAD,), jnp.float32),
        ],
        mesh=mesh,
        compiler_params=pltpu.CompilerParams(needs_layout_passes=False,
                                             use_tc_tiling_on_sc=False),
        scratch_types=[
            pltpu.VMEM((C,), jnp.int32),
            pltpu.VMEM((C,), jnp.int32),
            pltpu.VMEM((C * K,), jnp.int32),
            pltpu.VMEM((C, E), jnp.float32),
            pltpu.VMEM((C, E), jnp.float32),
            pltpu.VMEM((C * K, E), jnp.float32),
            pltpu.VMEM((C,), jnp.float32),
            pltpu.VMEM((C * KPAD,), jnp.float32),
            pltpu.SemaphoreType.DMA,
        ],
    )
    def sc_dots(center_hbm, context_hbm, negflat_hbm, inemb_hbm, outemb_hbm,
                pos_hbm, negdot_hbm,
                cidx, oidx, nidx, crow, orow, nrow, posv, negv, sem):
        wid = lax.axis_index("s") * info.num_cores + lax.axis_index("c")
        wbase = wid * rows_per_w
        lane = lax.iota(jnp.int32, LANES)

        def dot_rows(a_ref, a_row, b_ref, b_row):
            acc = a_ref[a_row, pl.ds(0, LANES)] * b_ref[b_row, pl.ds(0, LANES)]
            for v in range(1, EV):
                acc = acc + (a_ref[a_row, pl.ds(v * LANES, LANES)]
                             * b_ref[b_row, pl.ds(v * LANES, LANES)])
            return jnp.sum(acc, axis=0)

        def chunk_body(g, _):
            base = wbase + g * C

            pltpu.sync_copy(center_hbm.at[pl.ds(base, C)], cidx)
            pltpu.sync_copy(context_hbm.at[pl.ds(base, C)], oidx)
            pltpu.sync_copy(negflat_hbm.at[pl.ds(base * K, C * K)], nidx)

            copies = [
                pltpu.async_copy(inemb_hbm.at[cidx], crow, sem),
                pltpu.async_copy(outemb_hbm.at[oidx], orow, sem),
            ]
            for j in range(C * K // IDX_BLK):
                copies.append(pltpu.async_copy(
                    outemb_hbm.at[nidx.at[pl.ds(j * IDX_BLK, IDX_BLK)]],
                    nrow.at[pl.ds(j * IDX_BLK, IDX_BLK)],
                    sem))
            for cp in copies:
                cp.wait()

            def grp_body(grp, _):
                r0 = grp * LANES
                pvec = jnp.zeros((LANES,), jnp.float32)
                for i in range(LANES):
                    r = r0 + i
                    pvec = jnp.where(lane == i, dot_rows(crow, r, orow, r),
                                     pvec)
                    nvec0 = jnp.zeros((LANES,), jnp.float32)
                    nvec1 = jnp.zeros((LANES,), jnp.float32)
                    for k in range(K):
                        s = dot_rows(crow, r, nrow, r * K + k)
                        if k < LANES:
                            nvec0 = jnp.where(lane == k, s, nvec0)
                        else:
                            nvec1 = jnp.where(lane == (k - LANES), s, nvec1)
                    negv[pl.ds(r * KPAD, LANES)] = nvec0
                    negv[pl.ds(r * KPAD + LANES, LANES)] = nvec1
                posv[pl.ds(r0, LANES)] = pvec
                return 0

            lax.fori_loop(0, C // LANES, grp_body, 0)

            pltpu.sync_copy(posv, pos_hbm.at[pl.ds(base, C)])
            pltpu.sync_copy(negv, negdot_hbm.at[pl.ds(base * KPAD, C * KPAD)])
            return 0

        lax.fori_loop(0, n_chunks, chunk_body, 0)

    return sc_dots


def _make_loss_body(B, K):
    def loss_body(pos_ref, neg_ref, out_ref):
        pos = pos_ref[...]
        neg = neg_ref[...]
        ncols = neg.shape[1]
        k_of_col = jax.lax.broadcasted_iota(jnp.int32, neg.shape, 1) % KPAD
        pos_term = -jnp.log(jax.nn.sigmoid(pos) + 1e-09)
        neg_term = jnp.where(k_of_col < K,
                             -jnp.log(jax.nn.sigmoid(-neg) + 1e-09), 0.0)
        out_ref[0, 0] = (jnp.sum(pos_term) + jnp.sum(neg_term)) / B
    return loss_body


def kernel(center, context, negatives, input_emb, output_emb):
    B, = center.shape
    K = negatives.shape[1]
    V, E = input_emb.shape

    sc_dots = _make_sc_dots(B, K, E)
    pos_dot, neg_dot = sc_dots(
        center.astype(jnp.int32),
        context.astype(jnp.int32),
        negatives.reshape(B * K).astype(jnp.int32),
        input_emb,
        output_emb,
    )

    loss = pl.pallas_call(
        _make_loss_body(B, K),
        out_shape=jax.ShapeDtypeStruct((1, 1), jnp.float32),
        in_specs=[
            pl.BlockSpec(memory_space=pltpu.VMEM),
            pl.BlockSpec(memory_space=pltpu.VMEM),
        ],
        out_specs=pl.BlockSpec(memory_space=pltpu.SMEM),
    )(pos_dot.reshape(B // 128, 128), neg_dot.reshape(B * KPAD // 128, 128))
    return loss.reshape(())
